# baseline (speedup 1.0000x reference)
import functools

import jax
import jax.numpy as jnp
from jax import lax
from jax.experimental import pallas as pl
from jax.experimental.pallas import tpu as pltpu
from jax.experimental.pallas import tpu_sc as plsc

DIM = 1024
ROWS = 1000
BATCH = 16384
NUM_CORES = 2
NUM_SUBCORES = 16
NUM_WORKERS = NUM_CORES * NUM_SUBCORES
B_PER_W = BATCH // NUM_WORKERS
STAGE = 64


@jax.jit
def _gather(timestep, pe_matrix):
    mesh = plsc.VectorSubcoreMesh(
        core_axis_name="c", subcore_axis_name="s",
        num_cores=NUM_CORES, num_subcores=NUM_SUBCORES,
    )

    @functools.partial(
        pl.kernel,
        out_type=jax.ShapeDtypeStruct((BATCH, DIM), jnp.float32),
        mesh=mesh,
        scratch_types=[
            pltpu.VMEM_SHARED((ROWS, DIM), jnp.float32),
            pltpu.VMEM((B_PER_W,), jnp.int32),
        ],
    )
    def body(idx_hbm, table_hbm, out_hbm, table_sh, idx_v):
        cid = lax.axis_index("c")
        sid = lax.axis_index("s")
        wid = sid * NUM_CORES + cid
        base = wid * B_PER_W

        @pl.when(sid < NUM_SUBCORES - 1)
        def _():
            sl = pl.ds(sid * STAGE, STAGE)
            pltpu.sync_copy(table_hbm.at[sl], table_sh.at[sl])

        @pl.when(sid == NUM_SUBCORES - 1)
        def _():
            sl = pl.ds((NUM_SUBCORES - 1) * STAGE,
                       ROWS - (NUM_SUBCORES - 1) * STAGE)
            pltpu.sync_copy(table_hbm.at[sl], table_sh.at[sl])

        pltpu.sync_copy(idx_hbm.at[pl.ds(base, B_PER_W)], idx_v)
        plsc.subcore_barrier()

        def group(k, _):
            v = idx_v[pl.ds(k * 16, 16)]
            for e in range(16):
                r = v[e]
                pltpu.sync_copy(
                    table_sh.at[pl.ds(r, 1)],
                    out_hbm.at[pl.ds(base + k * 16 + e, 1)],
                )
            return 0

        lax.fori_loop(0, B_PER_W // 16, group, 0)

    return body(timestep, pe_matrix)


def kernel(timestep, pe_matrix):
    return _gather(timestep.astype(jnp.int32), pe_matrix)


# speedup vs baseline: 5.8640x; 5.8640x over previous
import functools

import jax
import jax.numpy as jnp
from jax import lax
from jax.experimental import pallas as pl
from jax.experimental.pallas import tpu as pltpu
from jax.experimental.pallas import tpu_sc as plsc

DIM = 1024
ROWS = 1000
BATCH = 16384
NUM_CORES = 2
NUM_SUBCORES = 16
NUM_WORKERS = NUM_CORES * NUM_SUBCORES
B_PER_W = BATCH // NUM_WORKERS
STAGE = 64


@jax.jit
def _gather(timestep, pe_matrix):
    mesh = plsc.VectorSubcoreMesh(
        core_axis_name="c", subcore_axis_name="s",
        num_cores=NUM_CORES, num_subcores=NUM_SUBCORES,
    )

    @functools.partial(
        pl.kernel,
        out_type=jax.ShapeDtypeStruct((BATCH, DIM), jnp.float32),
        mesh=mesh,
        scratch_types=[
            pltpu.VMEM_SHARED((ROWS, DIM), jnp.float32),
            pltpu.VMEM((B_PER_W,), jnp.int32),
            pltpu.SemaphoreType.DMA,
        ],
    )
    def body(idx_hbm, table_hbm, out_hbm, table_sh, idx_v, osem):
        cid = lax.axis_index("c")
        sid = lax.axis_index("s")
        wid = sid * NUM_CORES + cid
        base = wid * B_PER_W

        @pl.when(sid < NUM_SUBCORES - 1)
        def _():
            sl = pl.ds(sid * STAGE, STAGE)
            pltpu.sync_copy(table_hbm.at[sl], table_sh.at[sl])

        @pl.when(sid == NUM_SUBCORES - 1)
        def _():
            sl = pl.ds((NUM_SUBCORES - 1) * STAGE,
                       ROWS - (NUM_SUBCORES - 1) * STAGE)
            pltpu.sync_copy(table_hbm.at[sl], table_sh.at[sl])

        pltpu.sync_copy(idx_hbm.at[pl.ds(base, B_PER_W)], idx_v)
        plsc.subcore_barrier()

        def group(k, _):
            v = idx_v[pl.ds(k * 16, 16)]
            copies = []
            for e in range(16):
                r = v[e]
                copies.append(pltpu.async_copy(
                    table_sh.at[pl.ds(r, 1)],
                    out_hbm.at[pl.ds(base + k * 16 + e, 1)],
                    osem,
                ))
            for c in copies:
                c.wait()
            return 0

        lax.fori_loop(0, B_PER_W // 16, group, 0)

    return body(timestep, pe_matrix)


def kernel(timestep, pe_matrix):
    return _gather(timestep.astype(jnp.int32), pe_matrix)
